# Initial kernel scaffold; baseline (speedup 1.0000x reference)
import functools

import jax
import jax.numpy as jnp
from jax import lax
from jax.experimental import pallas as pl
from jax.experimental.pallas import tpu as pltpu
from jax.experimental.pallas import tpu_sc as plsc

HID = 128
N = 10000
NP = 10240
E = 320000
CB = 158
EPB = CB * 128 * 16
NI = 16384
EI = 262144
CI = 192
BINS = 2 * NP + 3 * NI
SD = BINS // 16
TRASH = 10001
DEGC = 349
DEG_EDGES = DEGC * 128 * 32

NC, NS = 2, 16


def _sc_mesh():
  return plsc.VectorSubcoreMesh(core_axis_name="c", subcore_axis_name="s",
                                num_cores=NC, num_subcores=NS)


def _deg_body(dstbins, zeros_hbm, out, idx_v, ones_v, acc_sh):
  c = lax.axis_index("c")
  s = lax.axis_index("s")
  wid = c * NS + s
  for t in range(8):
    ones_v[pl.ds(t * 16, 16)] = jnp.full((16,), 1.0, jnp.float32)
  pltpu.sync_copy(zeros_hbm, acc_sh.at[pl.ds(s * SD, SD)])
  pltpu.sync_copy(dstbins.at[pl.ds(wid * DEGC, DEGC)], idx_v)
  plsc.subcore_barrier()

  def body(j, carry):
    pltpu.sync_copy(ones_v, acc_sh.at[idx_v.at[j]], add=True)
    return carry

  lax.fori_loop(0, DEGC, body, 0)
  plsc.subcore_barrier()
  pltpu.sync_copy(acc_sh.at[pl.ds(s * SD, SD)],
                  out.at[pl.ds(c * BINS + s * SD, SD)])


_deg_kernel = functools.partial(
    pl.kernel,
    out_type=jax.ShapeDtypeStruct((2 * BINS,), jnp.float32),
    scratch_types=[
        pltpu.VMEM((DEGC, 128), jnp.int32),
        pltpu.VMEM((128,), jnp.float32),
        pltpu.VMEM_SHARED((BINS,), jnp.float32),
    ],
)


def _conv_body(g2, src, dst, out, srcv, dstv, rows0, rows1, sem0, sem1,
               acc_sh):
  c = lax.axis_index("c")
  s = lax.axis_index("s")
  stripe = NP // NS
  pltpu.sync_copy(g2.at[pl.ds(c * NP + s * stripe, stripe)],
                  acc_sh.at[pl.ds(s * stripe, stripe)])
  base = (c * NS + s) * CB
  pltpu.sync_copy(src.at[pl.ds(base, CB)], srcv)
  pltpu.sync_copy(dst.at[pl.ds(base, CB)], dstv)
  plsc.subcore_barrier()

  pltpu.async_copy(g2.at[srcv.at[0]], rows0, sem0)

  def body(i, carry):
    j = 2 * i
    pltpu.make_async_copy(g2.at[srcv.at[j]], rows0, sem0).wait()
    pltpu.async_copy(g2.at[srcv.at[j + 1]], rows1, sem1)
    pltpu.sync_copy(rows0, acc_sh.at[dstv.at[j]], add=True)
    pltpu.make_async_copy(g2.at[srcv.at[j + 1]], rows1, sem1).wait()
    pltpu.async_copy(g2.at[srcv.at[j + 2]], rows0, sem0)
    pltpu.sync_copy(rows1, acc_sh.at[dstv.at[j + 1]], add=True)
    return carry

  lax.fori_loop(0, CB // 2 - 1, body, 0)
  j = CB - 2
  pltpu.make_async_copy(g2.at[srcv.at[j]], rows0, sem0).wait()
  pltpu.async_copy(g2.at[srcv.at[j + 1]], rows1, sem1)
  pltpu.sync_copy(rows0, acc_sh.at[dstv.at[j]], add=True)
  pltpu.make_async_copy(g2.at[srcv.at[j + 1]], rows1, sem1).wait()
  pltpu.sync_copy(rows1, acc_sh.at[dstv.at[j + 1]], add=True)

  plsc.subcore_barrier()
  pltpu.sync_copy(acc_sh.at[pl.ds(s * stripe, stripe)],
                  out.at[pl.ds(c * NP + s * stripe, stripe)])


_conv_kernel = functools.partial(
    pl.kernel,
    out_type=jax.ShapeDtypeStruct((2 * NP, HID), jnp.float32),
    scratch_types=[
        pltpu.VMEM((CB, 128), jnp.int32),
        pltpu.VMEM((CB, 128), jnp.int32),
        pltpu.VMEM((128, HID), jnp.float32),
        pltpu.VMEM((128, HID), jnp.float32),
        pltpu.SemaphoreType.DMA,
        pltpu.SemaphoreType.DMA,
        pltpu.VMEM_SHARED((NP, HID), jnp.float32),
    ],
)


def _inter_body(g, src, dst, zeros_hbm, out, srcv, dstv, vals0, vals1,
                sem0, sem1, acc_sh):
  c = lax.axis_index("c")
  s = lax.axis_index("s")
  stripe = 3 * NI // NS
  pltpu.sync_copy(zeros_hbm.at[pl.ds(0, stripe)],
                  acc_sh.at[pl.ds(s * stripe, stripe)])
  base = (c * NS + s) * CI
  pltpu.sync_copy(src.at[pl.ds(base, CI)], srcv)
  pltpu.sync_copy(dst.at[pl.ds(base, CI)], dstv)
  plsc.subcore_barrier()

  pltpu.async_copy(g.at[srcv.at[0]], vals0, sem0)

  def body(i, carry):
    j = 2 * i
    pltpu.make_async_copy(g.at[srcv.at[j]], vals0, sem0).wait()
    pltpu.async_copy(g.at[srcv.at[j + 1]], vals1, sem1)
    pltpu.sync_copy(vals0, acc_sh.at[dstv.at[j]], add=True)
    pltpu.make_async_copy(g.at[srcv.at[j + 1]], vals1, sem1).wait()
    pltpu.async_copy(g.at[srcv.at[j + 2]], vals0, sem0)
    pltpu.sync_copy(vals1, acc_sh.at[dstv.at[j + 1]], add=True)
    return carry

  lax.fori_loop(0, CI // 2 - 1, body, 0)
  j = CI - 2
  pltpu.make_async_copy(g.at[srcv.at[j]], vals0, sem0).wait()
  pltpu.async_copy(g.at[srcv.at[j + 1]], vals1, sem1)
  pltpu.sync_copy(vals0, acc_sh.at[dstv.at[j]], add=True)
  pltpu.make_async_copy(g.at[srcv.at[j + 1]], vals1, sem1).wait()
  pltpu.sync_copy(vals1, acc_sh.at[dstv.at[j + 1]], add=True)

  plsc.subcore_barrier()
  pltpu.sync_copy(acc_sh.at[pl.ds(s * stripe, stripe)],
                  out.at[pl.ds(c * 3 * NI + s * stripe, stripe)])


_inter_kernel = functools.partial(
    pl.kernel,
    out_type=jax.ShapeDtypeStruct((2 * 3 * NI,), jnp.float32),
    scratch_types=[
        pltpu.VMEM((CI, 128), jnp.int32),
        pltpu.VMEM((CI, 128), jnp.int32),
        pltpu.VMEM((128,), jnp.float32),
        pltpu.VMEM((128,), jnp.float32),
        pltpu.SemaphoreType.DMA,
        pltpu.SemaphoreType.DMA,
        pltpu.VMEM_SHARED((3 * NI,), jnp.float32),
    ],
)


_BLK = 1280


def _tc_pre_body(x_ref, pdeg_ref, w_ref, o_ref):
  deg = pdeg_ref[0, :] + pdeg_ref[1, :] + 1.0
  dis = lax.rsqrt(deg)
  m = jnp.dot(x_ref[...], w_ref[...], preferred_element_type=jnp.float32)
  o_ref[...] = dis[:, None] * m


def _tc_mid_body(acc_ref, pdeg_ref, b_ref, w_ref, o_ref):
  deg = pdeg_ref[0, :] + pdeg_ref[1, :] + 1.0
  dis = lax.rsqrt(deg)
  h = jnp.maximum(dis[:, None] * acc_ref[...] + b_ref[...], 0.0)
  m = jnp.dot(h, w_ref[...], preferred_element_type=jnp.float32)
  o_ref[...] = dis[:, None] * m


def _tc_pool_body(acc_ref, pdeg_ref, pdegi_ref, b_ref, o_ref, sums_ref):
  i = pl.program_id(0)
  deg = pdeg_ref[0, :] + pdeg_ref[1, :] + 1.0
  dis = lax.rsqrt(deg)
  h = jnp.maximum(dis[:, None] * acc_ref[...] + b_ref[...], 0.0)
  lrow = (i % 8) * _BLK + lax.broadcasted_iota(jnp.int32, (_BLK, 1), 0)
  h = jnp.where(lrow < N, h, 0.0)
  sums_ref[pl.ds(i, 1), :] = jnp.sum(h, axis=0, keepdims=True)

  @pl.when(i == 15)
  def _():
    allsums = sums_ref[...]
    h1m = jnp.sum(allsums[0:8], axis=0) * (1.0 / N)
    h2m = jnp.sum(allsums[8:16], axis=0) * (1.0 / N)
    degi = pdegi_ref[0] + pdegi_ref[1] + 1.0
    disi = lax.rsqrt(degi)
    outer = h1m[:, None] * h2m[None, :]
    o_ref[...] = disi * outer[None, :, :]


def _tc_head_body(ps_ref, g_ref, pdegi_ref, wi_ref, bi_ref, fcw_ref,
                  fcb_ref, o_ref, acc_ref):
  i = pl.program_id(0)

  @pl.when(i == 0)
  def _():
    acc_ref[...] = jnp.zeros((8, HID), jnp.float32)

  degi = pdegi_ref[0] + pdegi_ref[1] + 1.0
  disi = lax.rsqrt(degi)
  s = disi * (ps_ref[0] + ps_ref[1] + g_ref[...])
  for k in range(3):
    sk = s[k, :]
    contrib = jnp.maximum(sk[:, None] * wi_ref[k, :][None, :]
                          + bi_ref[k, :][None, :], 0.0)
    acc_ref[k, :] = acc_ref[k, :] + jnp.sum(contrib, axis=0)

  @pl.when(i == 15)
  def _():
    v = (acc_ref[0, :] + acc_ref[1, :] + acc_ref[2, :]) * (1.0 / (3.0 * NI))
    o_ref[...] = jnp.dot(v[None, :], fcw_ref[...],
                         preferred_element_type=jnp.float32) + fcb_ref[...]


def _pad_edges(idx, pad_val, total):
  pad = jnp.full((total - idx.shape[0],), pad_val, jnp.int32)
  return jnp.concatenate([idx.astype(jnp.int32), pad])


def kernel(x_lig, x_tar, W1, b1, W2, b2, Wi1, bi1, Wi2, bi2, Wi3, bi3,
           fcW, fcb, lig_e_idx, tar_e_idx, inter_idx1, inter_idx2,
           inter_idx3):
  f32 = jnp.float32
  mesh = _sc_mesh()

  x = jnp.concatenate([
      jnp.pad(x_lig.astype(f32), ((0, NP - N), (0, 0))),
      jnp.pad(x_tar.astype(f32), ((0, NP - N), (0, 0))),
  ])

  lig_src = lig_e_idx[0].astype(jnp.int32)
  lig_dst = lig_e_idx[1].astype(jnp.int32)
  tar_src = tar_e_idx[0].astype(jnp.int32)
  tar_dst = tar_e_idx[1].astype(jnp.int32)
  i_src = [e[0].astype(jnp.int32) for e in (inter_idx1, inter_idx2,
                                            inter_idx3)]
  i_dst = [e[1].astype(jnp.int32) for e in (inter_idx1, inter_idx2,
                                            inter_idx3)]

  conv_src = jnp.concatenate([
      _pad_edges(lig_src, N, EPB),
      _pad_edges(tar_src + NP, NP + N, EPB),
  ]).reshape(2 * NS * CB, 128)
  conv_dst = jnp.concatenate([
      _pad_edges(lig_dst, N, EPB),
      _pad_edges(tar_dst, N, EPB),
  ]).reshape(2 * NS * CB, 128)

  deg_bins = jnp.concatenate([
      lig_dst, tar_dst + NP,
      i_dst[0] + 2 * NP, i_dst[1] + 2 * NP + NI, i_dst[2] + 2 * NP + 2 * NI,
      jnp.full((DEG_EDGES - 2 * E - 3 * EI,), TRASH, jnp.int32),
  ]).reshape(32 * DEGC, 128)

  int_src = jnp.concatenate(
      [i_src[k] + k * NI for k in range(3)]).reshape(32 * CI, 128)
  int_dst = jnp.concatenate(
      [i_dst[k] + k * NI for k in range(3)]).reshape(32 * CI, 128)

  zeros_hbm = jnp.zeros((SD,), f32)

  pdeg_flat = _deg_kernel(_deg_body, mesh=mesh)(deg_bins, zeros_hbm)
  pdeg = pdeg_flat.reshape(2, BINS)
  pdeg_nodes = pdeg[:, :2 * NP]
  pdeg_inter = pdeg[:, 2 * NP:].reshape(2, 3, NI)
  pdeg_inter4 = pdeg_inter.reshape(2, 3, 128, 128)

  grid16 = 16
  spec_x = pl.BlockSpec((_BLK, HID), lambda i: (i, 0))
  spec_pdeg = pl.BlockSpec((2, _BLK), lambda i: (0, i))
  spec_w = pl.BlockSpec((HID, HID), lambda i: (0, 0))
  spec_b = pl.BlockSpec((1, HID), lambda i: (0, 0))

  g1 = pl.pallas_call(
      _tc_pre_body,
      grid=(grid16,),
      in_specs=[spec_x, spec_pdeg, spec_w],
      out_specs=spec_x,
      out_shape=jax.ShapeDtypeStruct((2 * NP, HID), f32),
  )(x, pdeg_nodes, W1.astype(f32))

  acc1 = _conv_kernel(_conv_body, mesh=mesh)(g1, conv_src, conv_dst)

  g2 = pl.pallas_call(
      _tc_mid_body,
      grid=(grid16,),
      in_specs=[spec_x, spec_pdeg, spec_b, spec_w],
      out_specs=spec_x,
      out_shape=jax.ShapeDtypeStruct((2 * NP, HID), f32),
  )(acc1, pdeg_nodes, b1.astype(f32).reshape(1, HID), W2.astype(f32))

  acc2 = _conv_kernel(_conv_body, mesh=mesh)(g2, conv_src, conv_dst)

  g_all = pl.pallas_call(
      _tc_pool_body,
      grid=(grid16,),
      in_specs=[
          spec_x, spec_pdeg,
          pl.BlockSpec((2, 3, 128, 128), lambda i: (0, 0, 0, 0)),
          spec_b,
      ],
      out_specs=pl.BlockSpec((3, 128, 128), lambda i: (0, 0, 0)),
      out_shape=jax.ShapeDtypeStruct((3, 128, 128), f32),
      scratch_shapes=[pltpu.VMEM((16, HID), f32)],
  )(acc2, pdeg_nodes, pdeg_inter4, b2.astype(f32).reshape(1, HID))
  g_flat = g_all.reshape(3 * NI)

  ps_flat = _inter_kernel(_inter_body, mesh=mesh)(
      g_flat, int_src, int_dst, zeros_hbm)
  ps = ps_flat.reshape(2, 3, NI)

  wi = jnp.concatenate([Wi1, Wi2, Wi3]).astype(f32)
  bi = jnp.stack([bi1, bi2, bi3]).astype(f32)
  ib = 1024
  y = pl.pallas_call(
      _tc_head_body,
      grid=(grid16,),
      in_specs=[
          pl.BlockSpec((2, 3, ib), lambda i: (0, 0, i)),
          pl.BlockSpec((3, ib), lambda i: (0, i)),
          pl.BlockSpec((2, 3, ib), lambda i: (0, 0, i)),
          pl.BlockSpec((3, HID), lambda i: (0, 0)),
          pl.BlockSpec((3, HID), lambda i: (0, 0)),
          pl.BlockSpec((HID, HID), lambda i: (0, 0)),
          pl.BlockSpec((1, HID), lambda i: (0, 0)),
      ],
      out_specs=pl.BlockSpec((1, HID), lambda i: (0, 0)),
      out_shape=jax.ShapeDtypeStruct((1, HID), f32),
      scratch_shapes=[pltpu.VMEM((8, HID), f32)],
  )(ps, g_flat.reshape(3, NI), pdeg_inter, wi, bi, fcW.astype(f32),
    fcb.astype(f32).reshape(1, HID))
  return y


# speedup vs baseline: 23.5046x; 23.5046x over previous
import functools

import jax
import jax.numpy as jnp
from jax import lax
from jax.experimental import pallas as pl
from jax.experimental.pallas import tpu as pltpu
from jax.experimental.pallas import tpu_sc as plsc

HID = 128
N = 10000
NP = 10240
E = 320000
CB = 160
EPB = CB * 128 * 16
NI = 16384
EI = 262144
CI = 192
BINS = 2 * NP + 3 * NI
SD = BINS // 16
TRASH = 10001
DEGC = 352
DEG_EDGES = DEGC * 128 * 32

NC, NS = 2, 16


def _sc_mesh():
  return plsc.VectorSubcoreMesh(core_axis_name="c", subcore_axis_name="s",
                                num_cores=NC, num_subcores=NS)


def _deg_body(dstbins, zeros_hbm, out, idx_v, ones_v, acc_sh):
  c = lax.axis_index("c")
  s = lax.axis_index("s")
  wid = c * NS + s
  for t in range(8):
    ones_v[pl.ds(t * 16, 16)] = jnp.full((16,), 1.0, jnp.float32)
  pltpu.sync_copy(zeros_hbm, acc_sh.at[pl.ds(s * SD, SD)])
  pltpu.sync_copy(dstbins.at[pl.ds(wid * DEGC, DEGC)], idx_v)
  plsc.subcore_barrier()

  def body(j, carry):
    pltpu.sync_copy(ones_v, acc_sh.at[idx_v.at[j]], add=True)
    return carry

  lax.fori_loop(0, DEGC, body, 0)
  plsc.subcore_barrier()
  pltpu.sync_copy(acc_sh.at[pl.ds(s * SD, SD)],
                  out.at[pl.ds(c * BINS + s * SD, SD)])


_deg_kernel = functools.partial(
    pl.kernel,
    out_type=jax.ShapeDtypeStruct((2 * BINS,), jnp.float32),
    scratch_types=[
        pltpu.VMEM((DEGC, 128), jnp.int32),
        pltpu.VMEM((128,), jnp.float32),
        pltpu.VMEM_SHARED((BINS,), jnp.float32),
    ],
)


IG = 8
NG = CB // IG


def _conv_body(g2, src, dst, out, si0, di0, si1, di1, rows0, rows1,
               gsem0, gsem1, isem0, isem1, acc_sh):
  c = lax.axis_index("c")
  s = lax.axis_index("s")
  stripe = NP // NS
  base = (c * NS + s) * CB

  pltpu.sync_copy(src.at[pl.ds(base, IG)], si0)
  pltpu.sync_copy(dst.at[pl.ds(base, IG)], di0)
  pltpu.async_copy(src.at[pl.ds(base + IG, IG)], si1, isem1)
  pltpu.async_copy(dst.at[pl.ds(base + IG, IG)], di1, isem1)
  pltpu.sync_copy(g2.at[pl.ds(c * NP + s * stripe, stripe)],
                  acc_sh.at[pl.ds(s * stripe, stripe)])
  plsc.subcore_barrier()

  slots = ((si0, di0, isem0), (si1, di1, isem1))

  def sbody(m, carry):
    for q in (0, 1):
      g = 2 * m + q
      si_q, di_q, _ = slots[q]
      si_o, di_o, isem_o = slots[1 - q]
      _, _, isem_q = slots[q]
      pltpu.async_copy(g2.at[si_q.at[0]], rows0, gsem0)
      for k in range(IG):
        if k % 2 == 0:
          rbuf, rsem = rows0, gsem0
          nbuf, nsem = rows1, gsem1
        else:
          rbuf, rsem = rows1, gsem1
          nbuf, nsem = rows0, gsem0
        pltpu.make_async_copy(g2.at[si_q.at[k]], rbuf, rsem).wait()
        if k < IG - 1:
          pltpu.async_copy(g2.at[si_q.at[k + 1]], nbuf, nsem)
        else:
          nb = base + (g + 1) * IG
          pltpu.make_async_copy(src.at[pl.ds(nb, IG)], si_o, isem_o).wait()
          pltpu.make_async_copy(dst.at[pl.ds(nb, IG)], di_o, isem_o).wait()
          fb = base + (g + 2) * IG
          pltpu.async_copy(src.at[pl.ds(fb, IG)], si_q, isem_q)
          pltpu.async_copy(dst.at[pl.ds(fb, IG)], di_q, isem_q)
        pltpu.sync_copy(rbuf, acc_sh.at[di_q.at[k]], add=True)
    return carry

  lax.fori_loop(0, NG // 2, sbody, 0)
  nb = base + (NG + 1) * IG
  pltpu.make_async_copy(src.at[pl.ds(nb, IG)], si1, isem1).wait()
  pltpu.make_async_copy(dst.at[pl.ds(nb, IG)], di1, isem1).wait()

  plsc.subcore_barrier()
  pltpu.sync_copy(acc_sh.at[pl.ds(s * stripe, stripe)],
                  out.at[pl.ds(c * NP + s * stripe, stripe)])


_conv_kernel = functools.partial(
    pl.kernel,
    out_type=jax.ShapeDtypeStruct((2 * NP, HID), jnp.float32),
    scratch_types=[
        pltpu.VMEM((IG, 128), jnp.int32),
        pltpu.VMEM((IG, 128), jnp.int32),
        pltpu.VMEM((IG, 128), jnp.int32),
        pltpu.VMEM((IG, 128), jnp.int32),
        pltpu.VMEM((128, HID), jnp.float32),
        pltpu.VMEM((128, HID), jnp.float32),
        pltpu.SemaphoreType.DMA,
        pltpu.SemaphoreType.DMA,
        pltpu.SemaphoreType.DMA,
        pltpu.SemaphoreType.DMA,
        pltpu.VMEM_SHARED((NP, HID), jnp.float32),
    ],
)


def _inter_body(g, src, dst, zeros_hbm, out, srcv, dstv, vals0, vals1,
                sem0, sem1, acc_sh):
  c = lax.axis_index("c")
  s = lax.axis_index("s")
  stripe = 3 * NI // NS
  pltpu.sync_copy(zeros_hbm.at[pl.ds(0, stripe)],
                  acc_sh.at[pl.ds(s * stripe, stripe)])
  base = (c * NS + s) * CI
  pltpu.sync_copy(src.at[pl.ds(base, CI)], srcv)
  pltpu.sync_copy(dst.at[pl.ds(base, CI)], dstv)
  plsc.subcore_barrier()

  pltpu.async_copy(g.at[srcv.at[0]], vals0, sem0)

  def body(i, carry):
    j = 2 * i
    pltpu.make_async_copy(g.at[srcv.at[j]], vals0, sem0).wait()
    pltpu.async_copy(g.at[srcv.at[j + 1]], vals1, sem1)
    pltpu.sync_copy(vals0, acc_sh.at[dstv.at[j]], add=True)
    pltpu.make_async_copy(g.at[srcv.at[j + 1]], vals1, sem1).wait()
    pltpu.async_copy(g.at[srcv.at[j + 2]], vals0, sem0)
    pltpu.sync_copy(vals1, acc_sh.at[dstv.at[j + 1]], add=True)
    return carry

  lax.fori_loop(0, CI // 2 - 1, body, 0)
  j = CI - 2
  pltpu.make_async_copy(g.at[srcv.at[j]], vals0, sem0).wait()
  pltpu.async_copy(g.at[srcv.at[j + 1]], vals1, sem1)
  pltpu.sync_copy(vals0, acc_sh.at[dstv.at[j]], add=True)
  pltpu.make_async_copy(g.at[srcv.at[j + 1]], vals1, sem1).wait()
  pltpu.sync_copy(vals1, acc_sh.at[dstv.at[j + 1]], add=True)

  plsc.subcore_barrier()
  pltpu.sync_copy(acc_sh.at[pl.ds(s * stripe, stripe)],
                  out.at[pl.ds(c * 3 * NI + s * stripe, stripe)])


_inter_kernel = functools.partial(
    pl.kernel,
    out_type=jax.ShapeDtypeStruct((2 * 3 * NI,), jnp.float32),
    scratch_types=[
        pltpu.VMEM((CI, 128), jnp.int32),
        pltpu.VMEM((CI, 128), jnp.int32),
        pltpu.VMEM((128,), jnp.float32),
        pltpu.VMEM((128,), jnp.float32),
        pltpu.SemaphoreType.DMA,
        pltpu.SemaphoreType.DMA,
        pltpu.VMEM_SHARED((3 * NI,), jnp.float32),
    ],
)


_BLK = 1280


def _tc_pre_body(x_ref, pdeg_ref, w_ref, o_ref):
  deg = pdeg_ref[0, :] + pdeg_ref[1, :] + 1.0
  dis = lax.rsqrt(deg)
  m = jnp.dot(x_ref[...], w_ref[...], preferred_element_type=jnp.float32)
  o_ref[...] = dis[:, None] * m


def _tc_mid_body(acc_ref, pdeg_ref, b_ref, w_ref, o_ref):
  deg = pdeg_ref[0, :] + pdeg_ref[1, :] + 1.0
  dis = lax.rsqrt(deg)
  h = jnp.maximum(dis[:, None] * acc_ref[...] + b_ref[...], 0.0)
  m = jnp.dot(h, w_ref[...], preferred_element_type=jnp.float32)
  o_ref[...] = dis[:, None] * m


def _tc_pool_body(acc_ref, pdeg_ref, pdegi_ref, b_ref, o_ref, sums_ref):
  i = pl.program_id(0)
  deg = pdeg_ref[0, :] + pdeg_ref[1, :] + 1.0
  dis = lax.rsqrt(deg)
  h = jnp.maximum(dis[:, None] * acc_ref[...] + b_ref[...], 0.0)
  lrow = (i % 8) * _BLK + lax.broadcasted_iota(jnp.int32, (_BLK, 1), 0)
  h = jnp.where(lrow < N, h, 0.0)
  sums_ref[pl.ds(i, 1), :] = jnp.sum(h, axis=0, keepdims=True)

  @pl.when(i == 15)
  def _():
    allsums = sums_ref[...]
    h1m = jnp.sum(allsums[0:8], axis=0) * (1.0 / N)
    h2m = jnp.sum(allsums[8:16], axis=0) * (1.0 / N)
    degi = pdegi_ref[0] + pdegi_ref[1] + 1.0
    disi = lax.rsqrt(degi)
    outer = h1m[:, None] * h2m[None, :]
    o_ref[...] = disi * outer[None, :, :]


def _tc_head_body(ps_ref, g_ref, pdegi_ref, wi_ref, bi_ref, fcw_ref,
                  fcb_ref, o_ref, acc_ref):
  i = pl.program_id(0)

  @pl.when(i == 0)
  def _():
    acc_ref[...] = jnp.zeros((8, HID), jnp.float32)

  degi = pdegi_ref[0] + pdegi_ref[1] + 1.0
  disi = lax.rsqrt(degi)
  s = disi * (ps_ref[0] + ps_ref[1] + g_ref[...])
  for k in range(3):
    sk = s[k, :]
    contrib = jnp.maximum(sk[:, None] * wi_ref[k, :][None, :]
                          + bi_ref[k, :][None, :], 0.0)
    acc_ref[k, :] = acc_ref[k, :] + jnp.sum(contrib, axis=0)

  @pl.when(i == 15)
  def _():
    v = (acc_ref[0, :] + acc_ref[1, :] + acc_ref[2, :]) * (1.0 / (3.0 * NI))
    o_ref[...] = jnp.dot(v[None, :], fcw_ref[...],
                         preferred_element_type=jnp.float32) + fcb_ref[...]


def _pad_edges(idx, pad_val, total):
  pad = jnp.full((total - idx.shape[0],), pad_val, jnp.int32)
  return jnp.concatenate([idx.astype(jnp.int32), pad])


def kernel(x_lig, x_tar, W1, b1, W2, b2, Wi1, bi1, Wi2, bi2, Wi3, bi3,
           fcW, fcb, lig_e_idx, tar_e_idx, inter_idx1, inter_idx2,
           inter_idx3):
  f32 = jnp.float32
  mesh = _sc_mesh()

  x = jnp.concatenate([
      jnp.pad(x_lig.astype(f32), ((0, NP - N), (0, 0))),
      jnp.pad(x_tar.astype(f32), ((0, NP - N), (0, 0))),
  ])

  lig_src = lig_e_idx[0].astype(jnp.int32)
  lig_dst = lig_e_idx[1].astype(jnp.int32)
  tar_src = tar_e_idx[0].astype(jnp.int32)
  tar_dst = tar_e_idx[1].astype(jnp.int32)
  i_src = [e[0].astype(jnp.int32) for e in (inter_idx1, inter_idx2,
                                            inter_idx3)]
  i_dst = [e[1].astype(jnp.int32) for e in (inter_idx1, inter_idx2,
                                            inter_idx3)]

  conv_src = jnp.pad(jnp.concatenate([
      _pad_edges(lig_src, N, EPB),
      _pad_edges(tar_src + NP, NP + N, EPB),
  ]).reshape(2 * NS * CB, 128), ((0, 2 * IG), (0, 0)))
  conv_dst = jnp.pad(jnp.concatenate([
      _pad_edges(lig_dst, N, EPB),
      _pad_edges(tar_dst, N, EPB),
  ]).reshape(2 * NS * CB, 128), ((0, 2 * IG), (0, 0)))

  deg_bins = jnp.concatenate([
      lig_dst, tar_dst + NP,
      i_dst[0] + 2 * NP, i_dst[1] + 2 * NP + NI, i_dst[2] + 2 * NP + 2 * NI,
      jnp.full((DEG_EDGES - 2 * E - 3 * EI,), TRASH, jnp.int32),
  ]).reshape(32 * DEGC, 128)

  int_src = jnp.concatenate(
      [i_src[k] + k * NI for k in range(3)]).reshape(32 * CI, 128)
  int_dst = jnp.concatenate(
      [i_dst[k] + k * NI for k in range(3)]).reshape(32 * CI, 128)

  zeros_hbm = jnp.zeros((SD,), f32)

  pdeg_flat = _deg_kernel(_deg_body, mesh=mesh)(deg_bins, zeros_hbm)
  pdeg = pdeg_flat.reshape(2, BINS)
  pdeg_nodes = pdeg[:, :2 * NP]
  pdeg_inter = pdeg[:, 2 * NP:].reshape(2, 3, NI)
  pdeg_inter4 = pdeg_inter.reshape(2, 3, 128, 128)

  grid16 = 16
  spec_x = pl.BlockSpec((_BLK, HID), lambda i: (i, 0))
  spec_pdeg = pl.BlockSpec((2, _BLK), lambda i: (0, i))
  spec_w = pl.BlockSpec((HID, HID), lambda i: (0, 0))
  spec_b = pl.BlockSpec((1, HID), lambda i: (0, 0))

  g1 = pl.pallas_call(
      _tc_pre_body,
      grid=(grid16,),
      in_specs=[spec_x, spec_pdeg, spec_w],
      out_specs=spec_x,
      out_shape=jax.ShapeDtypeStruct((2 * NP, HID), f32),
  )(x, pdeg_nodes, W1.astype(f32))

  acc1 = _conv_kernel(_conv_body, mesh=mesh)(g1, conv_src, conv_dst)

  g2 = pl.pallas_call(
      _tc_mid_body,
      grid=(grid16,),
      in_specs=[spec_x, spec_pdeg, spec_b, spec_w],
      out_specs=spec_x,
      out_shape=jax.ShapeDtypeStruct((2 * NP, HID), f32),
  )(acc1, pdeg_nodes, b1.astype(f32).reshape(1, HID), W2.astype(f32))

  acc2 = _conv_kernel(_conv_body, mesh=mesh)(g2, conv_src, conv_dst)

  g_all = pl.pallas_call(
      _tc_pool_body,
      grid=(grid16,),
      in_specs=[
          spec_x, spec_pdeg,
          pl.BlockSpec((2, 3, 128, 128), lambda i: (0, 0, 0, 0)),
          spec_b,
      ],
      out_specs=pl.BlockSpec((3, 128, 128), lambda i: (0, 0, 0)),
      out_shape=jax.ShapeDtypeStruct((3, 128, 128), f32),
      scratch_shapes=[pltpu.VMEM((16, HID), f32)],
  )(acc2, pdeg_nodes, pdeg_inter4, b2.astype(f32).reshape(1, HID))
  g_flat = g_all.reshape(3 * NI)

  ps_flat = _inter_kernel(_inter_body, mesh=mesh)(
      g_flat, int_src, int_dst, zeros_hbm)
  ps = ps_flat.reshape(2, 3, NI)

  wi = jnp.concatenate([Wi1, Wi2, Wi3]).astype(f32)
  bi = jnp.stack([bi1, bi2, bi3]).astype(f32)
  ib = 1024
  y = pl.pallas_call(
      _tc_head_body,
      grid=(grid16,),
      in_specs=[
          pl.BlockSpec((2, 3, ib), lambda i: (0, 0, i)),
          pl.BlockSpec((3, ib), lambda i: (0, i)),
          pl.BlockSpec((2, 3, ib), lambda i: (0, 0, i)),
          pl.BlockSpec((3, HID), lambda i: (0, 0)),
          pl.BlockSpec((3, HID), lambda i: (0, 0)),
          pl.BlockSpec((HID, HID), lambda i: (0, 0)),
          pl.BlockSpec((1, HID), lambda i: (0, 0)),
      ],
      out_specs=pl.BlockSpec((1, HID), lambda i: (0, 0)),
      out_shape=jax.ShapeDtypeStruct((1, HID), f32),
      scratch_shapes=[pltpu.VMEM((8, HID), f32)],
  )(ps, g_flat.reshape(3, NI), pdeg_inter, wi, bi, fcW.astype(f32),
    fcb.astype(f32).reshape(1, HID))
  return y


# speedup vs baseline: 23.5963x; 1.0039x over previous
import functools

import jax
import jax.numpy as jnp
from jax import lax
from jax.experimental import pallas as pl
from jax.experimental.pallas import tpu as pltpu
from jax.experimental.pallas import tpu_sc as plsc

HID = 128
N = 10000
NP = 10240
E = 320000
CB = 160
EPB = CB * 128 * 16
NI = 16384
EI = 262144
CI = 192
BINS = 2 * NP + 3 * NI
SD = BINS // 16
TRASH = 10001
DEGC = 352
DEG_EDGES = DEGC * 128 * 32

NC, NS = 2, 16


def _sc_mesh():
  return plsc.VectorSubcoreMesh(core_axis_name="c", subcore_axis_name="s",
                                num_cores=NC, num_subcores=NS)


def _deg_body(dstbins, zeros_hbm, out, idx_v, ones_v, acc_sh):
  c = lax.axis_index("c")
  s = lax.axis_index("s")
  wid = c * NS + s
  for t in range(8):
    ones_v[pl.ds(t * 16, 16)] = jnp.full((16,), 1.0, jnp.float32)
  pltpu.sync_copy(zeros_hbm, acc_sh.at[pl.ds(s * SD, SD)])
  pltpu.sync_copy(dstbins.at[pl.ds(wid * DEGC, DEGC)], idx_v)
  plsc.subcore_barrier()

  def body(j, carry):
    pltpu.sync_copy(ones_v, acc_sh.at[idx_v.at[j]], add=True)
    return carry

  lax.fori_loop(0, DEGC, body, 0)
  plsc.subcore_barrier()
  pltpu.sync_copy(acc_sh.at[pl.ds(s * SD, SD)],
                  out.at[pl.ds(c * BINS + s * SD, SD)])


_deg_kernel = functools.partial(
    pl.kernel,
    out_type=jax.ShapeDtypeStruct((2 * BINS,), jnp.float32),
    scratch_types=[
        pltpu.VMEM((DEGC, 128), jnp.int32),
        pltpu.VMEM((128,), jnp.float32),
        pltpu.VMEM_SHARED((BINS,), jnp.float32),
    ],
)


IG = 8
NG = CB // IG


def _conv_body(g2, src, dst, out, si0, di0, si1, di1, rows0, rows1, dumidx,
               gsem0, gsem1, isem0, isem1, ssem0, ssem1, acc_sh):
  c = lax.axis_index("c")
  s = lax.axis_index("s")
  stripe = NP // NS
  base = (c * NS + s) * CB

  pltpu.sync_copy(src.at[pl.ds(base, IG)], si0)
  pltpu.sync_copy(dst.at[pl.ds(base, IG)], di0)
  pltpu.async_copy(src.at[pl.ds(base + IG, IG)], si1, isem1)
  pltpu.async_copy(dst.at[pl.ds(base + IG, IG)], di1, isem1)
  for t in range(8):
    dumidx[pl.ds(t * 16, 16)] = jnp.full((16,), N, jnp.int32)
  pltpu.sync_copy(g2.at[pl.ds(c * NP + s * stripe, stripe)],
                  acc_sh.at[pl.ds(s * stripe, stripe)])
  plsc.subcore_barrier()

  pltpu.async_copy(g2.at[si0.at[0]], rows0, gsem0)
  pltpu.async_copy(rows1, acc_sh.at[dumidx], ssem1, add=True)

  slots = ((si0, di0, isem0), (si1, di1, isem1))

  def sbody(m, carry):
    for q in (0, 1):
      g = 2 * m + q
      si_q, di_q, isem_q = slots[q]
      si_o, di_o, isem_o = slots[1 - q]
      for k in range(IG):
        if k % 2 == 0:
          rbuf, rsem, ssem = rows0, gsem0, ssem0
          nbuf, nrsem, nssem = rows1, gsem1, ssem1
        else:
          rbuf, rsem, ssem = rows1, gsem1, ssem1
          nbuf, nrsem, nssem = rows0, gsem0, ssem0
        pltpu.make_async_copy(g2.at[si_q.at[k]], rbuf, rsem).wait()
        pltpu.async_copy(rbuf, acc_sh.at[di_q.at[k]], ssem, add=True)
        pltpu.make_async_copy(nbuf, acc_sh.at[di_q.at[k]], nssem).wait()
        if k < IG - 1:
          pltpu.async_copy(g2.at[si_q.at[k + 1]], nbuf, nrsem)
        else:
          nb = base + (g + 1) * IG
          pltpu.make_async_copy(src.at[pl.ds(nb, IG)], si_o, isem_o).wait()
          pltpu.make_async_copy(dst.at[pl.ds(nb, IG)], di_o, isem_o).wait()
          fb = base + (g + 2) * IG
          pltpu.async_copy(src.at[pl.ds(fb, IG)], si_q, isem_q)
          pltpu.async_copy(dst.at[pl.ds(fb, IG)], di_q, isem_q)
          pltpu.async_copy(g2.at[si_o.at[0]], nbuf, nrsem)
    return carry

  lax.fori_loop(0, NG // 2, sbody, 0)
  pltpu.make_async_copy(g2.at[si0.at[0]], rows0, gsem0).wait()
  pltpu.make_async_copy(rows1, acc_sh.at[di1.at[0]], ssem1).wait()
  nb = base + (NG + 1) * IG
  pltpu.make_async_copy(src.at[pl.ds(nb, IG)], si1, isem1).wait()
  pltpu.make_async_copy(dst.at[pl.ds(nb, IG)], di1, isem1).wait()

  plsc.subcore_barrier()
  pltpu.sync_copy(acc_sh.at[pl.ds(s * stripe, stripe)],
                  out.at[pl.ds(c * NP + s * stripe, stripe)])


_conv_kernel = functools.partial(
    pl.kernel,
    out_type=jax.ShapeDtypeStruct((2 * NP, HID), jnp.float32),
    scratch_types=[
        pltpu.VMEM((IG, 128), jnp.int32),
        pltpu.VMEM((IG, 128), jnp.int32),
        pltpu.VMEM((IG, 128), jnp.int32),
        pltpu.VMEM((IG, 128), jnp.int32),
        pltpu.VMEM((128, HID), jnp.float32),
        pltpu.VMEM((128, HID), jnp.float32),
        pltpu.VMEM((128,), jnp.int32),
        pltpu.SemaphoreType.DMA,
        pltpu.SemaphoreType.DMA,
        pltpu.SemaphoreType.DMA,
        pltpu.SemaphoreType.DMA,
        pltpu.SemaphoreType.DMA,
        pltpu.SemaphoreType.DMA,
        pltpu.VMEM_SHARED((NP, HID), jnp.float32),
    ],
)


def _inter_body(g, src, dst, zeros_hbm, out, srcv, dstv, vals0, vals1,
                sem0, sem1, acc_sh):
  c = lax.axis_index("c")
  s = lax.axis_index("s")
  stripe = 3 * NI // NS
  pltpu.sync_copy(zeros_hbm.at[pl.ds(0, stripe)],
                  acc_sh.at[pl.ds(s * stripe, stripe)])
  base = (c * NS + s) * CI
  pltpu.sync_copy(src.at[pl.ds(base, CI)], srcv)
  pltpu.sync_copy(dst.at[pl.ds(base, CI)], dstv)
  plsc.subcore_barrier()

  pltpu.async_copy(g.at[srcv.at[0]], vals0, sem0)

  def body(i, carry):
    j = 2 * i
    pltpu.make_async_copy(g.at[srcv.at[j]], vals0, sem0).wait()
    pltpu.async_copy(g.at[srcv.at[j + 1]], vals1, sem1)
    pltpu.sync_copy(vals0, acc_sh.at[dstv.at[j]], add=True)
    pltpu.make_async_copy(g.at[srcv.at[j + 1]], vals1, sem1).wait()
    pltpu.async_copy(g.at[srcv.at[j + 2]], vals0, sem0)
    pltpu.sync_copy(vals1, acc_sh.at[dstv.at[j + 1]], add=True)
    return carry

  lax.fori_loop(0, CI // 2 - 1, body, 0)
  j = CI - 2
  pltpu.make_async_copy(g.at[srcv.at[j]], vals0, sem0).wait()
  pltpu.async_copy(g.at[srcv.at[j + 1]], vals1, sem1)
  pltpu.sync_copy(vals0, acc_sh.at[dstv.at[j]], add=True)
  pltpu.make_async_copy(g.at[srcv.at[j + 1]], vals1, sem1).wait()
  pltpu.sync_copy(vals1, acc_sh.at[dstv.at[j + 1]], add=True)

  plsc.subcore_barrier()
  pltpu.sync_copy(acc_sh.at[pl.ds(s * stripe, stripe)],
                  out.at[pl.ds(c * 3 * NI + s * stripe, stripe)])


_inter_kernel = functools.partial(
    pl.kernel,
    out_type=jax.ShapeDtypeStruct((2 * 3 * NI,), jnp.float32),
    scratch_types=[
        pltpu.VMEM((CI, 128), jnp.int32),
        pltpu.VMEM((CI, 128), jnp.int32),
        pltpu.VMEM((128,), jnp.float32),
        pltpu.VMEM((128,), jnp.float32),
        pltpu.SemaphoreType.DMA,
        pltpu.SemaphoreType.DMA,
        pltpu.VMEM_SHARED((3 * NI,), jnp.float32),
    ],
)


_BLK = 1280


def _tc_pre_body(x_ref, pdeg_ref, w_ref, o_ref):
  deg = pdeg_ref[0, :] + pdeg_ref[1, :] + 1.0
  dis = lax.rsqrt(deg)
  m = jnp.dot(x_ref[...], w_ref[...], preferred_element_type=jnp.float32)
  o_ref[...] = dis[:, None] * m


def _tc_mid_body(acc_ref, pdeg_ref, b_ref, w_ref, o_ref):
  deg = pdeg_ref[0, :] + pdeg_ref[1, :] + 1.0
  dis = lax.rsqrt(deg)
  h = jnp.maximum(dis[:, None] * acc_ref[...] + b_ref[...], 0.0)
  m = jnp.dot(h, w_ref[...], preferred_element_type=jnp.float32)
  o_ref[...] = dis[:, None] * m


def _tc_pool_body(acc_ref, pdeg_ref, pdegi_ref, b_ref, o_ref, sums_ref):
  i = pl.program_id(0)
  deg = pdeg_ref[0, :] + pdeg_ref[1, :] + 1.0
  dis = lax.rsqrt(deg)
  h = jnp.maximum(dis[:, None] * acc_ref[...] + b_ref[...], 0.0)
  lrow = (i % 8) * _BLK + lax.broadcasted_iota(jnp.int32, (_BLK, 1), 0)
  h = jnp.where(lrow < N, h, 0.0)
  sums_ref[pl.ds(i, 1), :] = jnp.sum(h, axis=0, keepdims=True)

  @pl.when(i == 15)
  def _():
    allsums = sums_ref[...]
    h1m = jnp.sum(allsums[0:8], axis=0) * (1.0 / N)
    h2m = jnp.sum(allsums[8:16], axis=0) * (1.0 / N)
    degi = pdegi_ref[0] + pdegi_ref[1] + 1.0
    disi = lax.rsqrt(degi)
    outer = h1m[:, None] * h2m[None, :]
    o_ref[...] = disi * outer[None, :, :]


def _tc_head_body(ps_ref, g_ref, pdegi_ref, wi_ref, bi_ref, fcw_ref,
                  fcb_ref, o_ref, acc_ref):
  i = pl.program_id(0)

  @pl.when(i == 0)
  def _():
    acc_ref[...] = jnp.zeros((8, HID), jnp.float32)

  degi = pdegi_ref[0] + pdegi_ref[1] + 1.0
  disi = lax.rsqrt(degi)
  s = disi * (ps_ref[0] + ps_ref[1] + g_ref[...])
  for k in range(3):
    sk = s[k, :]
    contrib = jnp.maximum(sk[:, None] * wi_ref[k, :][None, :]
                          + bi_ref[k, :][None, :], 0.0)
    acc_ref[k, :] = acc_ref[k, :] + jnp.sum(contrib, axis=0)

  @pl.when(i == 15)
  def _():
    v = (acc_ref[0, :] + acc_ref[1, :] + acc_ref[2, :]) * (1.0 / (3.0 * NI))
    o_ref[...] = jnp.dot(v[None, :], fcw_ref[...],
                         preferred_element_type=jnp.float32) + fcb_ref[...]


def _pad_edges(idx, pad_val, total):
  pad = jnp.full((total - idx.shape[0],), pad_val, jnp.int32)
  return jnp.concatenate([idx.astype(jnp.int32), pad])


def kernel(x_lig, x_tar, W1, b1, W2, b2, Wi1, bi1, Wi2, bi2, Wi3, bi3,
           fcW, fcb, lig_e_idx, tar_e_idx, inter_idx1, inter_idx2,
           inter_idx3):
  f32 = jnp.float32
  mesh = _sc_mesh()

  x = jnp.concatenate([
      jnp.pad(x_lig.astype(f32), ((0, NP - N), (0, 0))),
      jnp.pad(x_tar.astype(f32), ((0, NP - N), (0, 0))),
  ])

  lig_src = lig_e_idx[0].astype(jnp.int32)
  lig_dst = lig_e_idx[1].astype(jnp.int32)
  tar_src = tar_e_idx[0].astype(jnp.int32)
  tar_dst = tar_e_idx[1].astype(jnp.int32)
  i_src = [e[0].astype(jnp.int32) for e in (inter_idx1, inter_idx2,
                                            inter_idx3)]
  i_dst = [e[1].astype(jnp.int32) for e in (inter_idx1, inter_idx2,
                                            inter_idx3)]

  conv_src = jnp.pad(jnp.concatenate([
      _pad_edges(lig_src, N, EPB),
      _pad_edges(tar_src + NP, NP + N, EPB),
  ]).reshape(2 * NS * CB, 128), ((0, 2 * IG), (0, 0)))
  conv_dst = jnp.pad(jnp.concatenate([
      _pad_edges(lig_dst, N, EPB),
      _pad_edges(tar_dst, N, EPB),
  ]).reshape(2 * NS * CB, 128), ((0, 2 * IG), (0, 0)))

  deg_bins = jnp.concatenate([
      lig_dst, tar_dst + NP,
      i_dst[0] + 2 * NP, i_dst[1] + 2 * NP + NI, i_dst[2] + 2 * NP + 2 * NI,
      jnp.full((DEG_EDGES - 2 * E - 3 * EI,), TRASH, jnp.int32),
  ]).reshape(32 * DEGC, 128)

  int_src = jnp.concatenate(
      [i_src[k] + k * NI for k in range(3)]).reshape(32 * CI, 128)
  int_dst = jnp.concatenate(
      [i_dst[k] + k * NI for k in range(3)]).reshape(32 * CI, 128)

  zeros_hbm = jnp.zeros((SD,), f32)

  pdeg_flat = _deg_kernel(_deg_body, mesh=mesh)(deg_bins, zeros_hbm)
  pdeg = pdeg_flat.reshape(2, BINS)
  pdeg_nodes = pdeg[:, :2 * NP]
  pdeg_inter = pdeg[:, 2 * NP:].reshape(2, 3, NI)
  pdeg_inter4 = pdeg_inter.reshape(2, 3, 128, 128)

  grid16 = 16
  spec_x = pl.BlockSpec((_BLK, HID), lambda i: (i, 0))
  spec_pdeg = pl.BlockSpec((2, _BLK), lambda i: (0, i))
  spec_w = pl.BlockSpec((HID, HID), lambda i: (0, 0))
  spec_b = pl.BlockSpec((1, HID), lambda i: (0, 0))

  g1 = pl.pallas_call(
      _tc_pre_body,
      grid=(grid16,),
      in_specs=[spec_x, spec_pdeg, spec_w],
      out_specs=spec_x,
      out_shape=jax.ShapeDtypeStruct((2 * NP, HID), f32),
  )(x, pdeg_nodes, W1.astype(f32))

  acc1 = _conv_kernel(_conv_body, mesh=mesh)(g1, conv_src, conv_dst)

  g2 = pl.pallas_call(
      _tc_mid_body,
      grid=(grid16,),
      in_specs=[spec_x, spec_pdeg, spec_b, spec_w],
      out_specs=spec_x,
      out_shape=jax.ShapeDtypeStruct((2 * NP, HID), f32),
  )(acc1, pdeg_nodes, b1.astype(f32).reshape(1, HID), W2.astype(f32))

  acc2 = _conv_kernel(_conv_body, mesh=mesh)(g2, conv_src, conv_dst)

  g_all = pl.pallas_call(
      _tc_pool_body,
      grid=(grid16,),
      in_specs=[
          spec_x, spec_pdeg,
          pl.BlockSpec((2, 3, 128, 128), lambda i: (0, 0, 0, 0)),
          spec_b,
      ],
      out_specs=pl.BlockSpec((3, 128, 128), lambda i: (0, 0, 0)),
      out_shape=jax.ShapeDtypeStruct((3, 128, 128), f32),
      scratch_shapes=[pltpu.VMEM((16, HID), f32)],
  )(acc2, pdeg_nodes, pdeg_inter4, b2.astype(f32).reshape(1, HID))
  g_flat = g_all.reshape(3 * NI)

  ps_flat = _inter_kernel(_inter_body, mesh=mesh)(
      g_flat, int_src, int_dst, zeros_hbm)
  ps = ps_flat.reshape(2, 3, NI)

  wi = jnp.concatenate([Wi1, Wi2, Wi3]).astype(f32)
  bi = jnp.stack([bi1, bi2, bi3]).astype(f32)
  ib = 1024
  y = pl.pallas_call(
      _tc_head_body,
      grid=(grid16,),
      in_specs=[
          pl.BlockSpec((2, 3, ib), lambda i: (0, 0, i)),
          pl.BlockSpec((3, ib), lambda i: (0, i)),
          pl.BlockSpec((2, 3, ib), lambda i: (0, 0, i)),
          pl.BlockSpec((3, HID), lambda i: (0, 0)),
          pl.BlockSpec((3, HID), lambda i: (0, 0)),
          pl.BlockSpec((HID, HID), lambda i: (0, 0)),
          pl.BlockSpec((1, HID), lambda i: (0, 0)),
      ],
      out_specs=pl.BlockSpec((1, HID), lambda i: (0, 0)),
      out_shape=jax.ShapeDtypeStruct((1, HID), f32),
      scratch_shapes=[pltpu.VMEM((8, HID), f32)],
  )(ps, g_flat.reshape(3, NI), pdeg_inter, wi, bi, fcW.astype(f32),
    fcb.astype(f32).reshape(1, HID))
  return y


# speedup vs baseline: 30.7832x; 1.3046x over previous
import functools

import jax
import jax.numpy as jnp
from jax import lax
from jax.experimental import pallas as pl
from jax.experimental.pallas import tpu as pltpu
from jax.experimental.pallas import tpu_sc as plsc

HID = 128
N = 10000
NP = 10240
E = 320000
CB = 160
EPB = CB * 128 * 16
NI = 16384
EI = 262144
CI = 192
BINS = 2 * NP + 3 * NI
SD = BINS // 16
TRASH = 10001
DEGC = 352
DEG_EDGES = DEGC * 128 * 32

NC, NS = 2, 16


def _sc_mesh():
  return plsc.VectorSubcoreMesh(core_axis_name="c", subcore_axis_name="s",
                                num_cores=NC, num_subcores=NS)


def _deg_body(dstbins, zeros_hbm, out, idx_v, ones_v, acc_sh):
  c = lax.axis_index("c")
  s = lax.axis_index("s")
  wid = c * NS + s
  for t in range(8):
    ones_v[pl.ds(t * 16, 16)] = jnp.full((16,), 1.0, jnp.float32)
  pltpu.sync_copy(zeros_hbm, acc_sh.at[pl.ds(s * SD, SD)])
  pltpu.sync_copy(dstbins.at[pl.ds(wid * DEGC, DEGC)], idx_v)
  plsc.subcore_barrier()

  def body(j, carry):
    pltpu.sync_copy(ones_v, acc_sh.at[idx_v.at[j]], add=True)
    return carry

  lax.fori_loop(0, DEGC, body, 0)
  plsc.subcore_barrier()
  pltpu.sync_copy(acc_sh.at[pl.ds(s * SD, SD)],
                  out.at[pl.ds(c * BINS + s * SD, SD)])


_deg_kernel = functools.partial(
    pl.kernel,
    out_type=jax.ShapeDtypeStruct((2 * BINS,), jnp.float32),
    scratch_types=[
        pltpu.VMEM((DEGC, 128), jnp.int32),
        pltpu.VMEM((128,), jnp.float32),
        pltpu.VMEM_SHARED((BINS,), jnp.float32),
    ],
)


IG = 8
NG = CB // IG


def _conv_body(g2, src, dst, out, si0, di0, si1, di1, rows0, rows1, dumidx,
               gsem0, gsem1, isem0, isem1, ssem0, ssem1, acc_sh):
  c = lax.axis_index("c")
  s = lax.axis_index("s")
  stripe = NP // NS
  base = (c * NS + s) * CB

  pltpu.sync_copy(src.at[pl.ds(base, IG)], si0)
  pltpu.sync_copy(dst.at[pl.ds(base, IG)], di0)
  pltpu.async_copy(src.at[pl.ds(base + IG, IG)], si1, isem1)
  pltpu.async_copy(dst.at[pl.ds(base + IG, IG)], di1, isem1)
  for t in range(8):
    dumidx[pl.ds(t * 16, 16)] = jnp.full((16,), N, jnp.int32)
  pltpu.sync_copy(g2.at[pl.ds(c * NP + s * stripe, stripe)],
                  acc_sh.at[pl.ds(s * stripe, stripe)])
  plsc.subcore_barrier()

  pltpu.async_copy(g2.at[pl.ds(c * NP, 128)], rows0, gsem0)
  pltpu.async_copy(rows1, acc_sh.at[dumidx], ssem1, add=True)

  slots = ((si0, di0, isem0), (si1, di1, isem1))

  def sbody(m, carry):
    for q in (0, 1):
      g = 2 * m + q
      si_q, di_q, isem_q = slots[q]
      si_o, di_o, isem_o = slots[1 - q]
      for k in range(IG):
        if k % 2 == 0:
          rbuf, rsem, ssem = rows0, gsem0, ssem0
          nbuf, nrsem, nssem = rows1, gsem1, ssem1
        else:
          rbuf, rsem, ssem = rows1, gsem1, ssem1
          nbuf, nrsem, nssem = rows0, gsem0, ssem0
        pltpu.make_async_copy(g2.at[si_q.at[k]], rbuf, rsem).wait()
        pltpu.async_copy(rbuf, acc_sh.at[pl.ds(s * stripe, 128)], ssem)
        pltpu.make_async_copy(nbuf, acc_sh.at[pl.ds(s * stripe, 128)],
                              nssem).wait()
        if k < IG - 1:
          pltpu.async_copy(g2.at[pl.ds(c * NP, 128)], nbuf, nrsem)
        else:
          nb = base + (g + 1) * IG
          pltpu.make_async_copy(src.at[pl.ds(nb, IG)], si_o, isem_o).wait()
          pltpu.make_async_copy(dst.at[pl.ds(nb, IG)], di_o, isem_o).wait()
          fb = base + (g + 2) * IG
          pltpu.async_copy(src.at[pl.ds(fb, IG)], si_q, isem_q)
          pltpu.async_copy(dst.at[pl.ds(fb, IG)], di_q, isem_q)
          pltpu.async_copy(g2.at[pl.ds(c * NP, 128)], nbuf, nrsem)
    return carry

  lax.fori_loop(0, NG // 2, sbody, 0)
  pltpu.make_async_copy(g2.at[si0.at[0]], rows0, gsem0).wait()
  pltpu.make_async_copy(rows1, acc_sh.at[di1.at[0]], ssem1).wait()
  nb = base + (NG + 1) * IG
  pltpu.make_async_copy(src.at[pl.ds(nb, IG)], si1, isem1).wait()
  pltpu.make_async_copy(dst.at[pl.ds(nb, IG)], di1, isem1).wait()

  plsc.subcore_barrier()
  pltpu.sync_copy(acc_sh.at[pl.ds(s * stripe, stripe)],
                  out.at[pl.ds(c * NP + s * stripe, stripe)])


_conv_kernel = functools.partial(
    pl.kernel,
    out_type=jax.ShapeDtypeStruct((2 * NP, HID), jnp.float32),
    scratch_types=[
        pltpu.VMEM((IG, 128), jnp.int32),
        pltpu.VMEM((IG, 128), jnp.int32),
        pltpu.VMEM((IG, 128), jnp.int32),
        pltpu.VMEM((IG, 128), jnp.int32),
        pltpu.VMEM((128, HID), jnp.float32),
        pltpu.VMEM((128, HID), jnp.float32),
        pltpu.VMEM((128,), jnp.int32),
        pltpu.SemaphoreType.DMA,
        pltpu.SemaphoreType.DMA,
        pltpu.SemaphoreType.DMA,
        pltpu.SemaphoreType.DMA,
        pltpu.SemaphoreType.DMA,
        pltpu.SemaphoreType.DMA,
        pltpu.VMEM_SHARED((NP, HID), jnp.float32),
    ],
)


def _inter_body(g, src, dst, zeros_hbm, out, srcv, dstv, vals0, vals1,
                sem0, sem1, acc_sh):
  c = lax.axis_index("c")
  s = lax.axis_index("s")
  stripe = 3 * NI // NS
  pltpu.sync_copy(zeros_hbm.at[pl.ds(0, stripe)],
                  acc_sh.at[pl.ds(s * stripe, stripe)])
  base = (c * NS + s) * CI
  pltpu.sync_copy(src.at[pl.ds(base, CI)], srcv)
  pltpu.sync_copy(dst.at[pl.ds(base, CI)], dstv)
  plsc.subcore_barrier()

  pltpu.async_copy(g.at[srcv.at[0]], vals0, sem0)

  def body(i, carry):
    j = 2 * i
    pltpu.make_async_copy(g.at[srcv.at[j]], vals0, sem0).wait()
    pltpu.async_copy(g.at[srcv.at[j + 1]], vals1, sem1)
    pltpu.sync_copy(vals0, acc_sh.at[dstv.at[j]], add=True)
    pltpu.make_async_copy(g.at[srcv.at[j + 1]], vals1, sem1).wait()
    pltpu.async_copy(g.at[srcv.at[j + 2]], vals0, sem0)
    pltpu.sync_copy(vals1, acc_sh.at[dstv.at[j + 1]], add=True)
    return carry

  lax.fori_loop(0, CI // 2 - 1, body, 0)
  j = CI - 2
  pltpu.make_async_copy(g.at[srcv.at[j]], vals0, sem0).wait()
  pltpu.async_copy(g.at[srcv.at[j + 1]], vals1, sem1)
  pltpu.sync_copy(vals0, acc_sh.at[dstv.at[j]], add=True)
  pltpu.make_async_copy(g.at[srcv.at[j + 1]], vals1, sem1).wait()
  pltpu.sync_copy(vals1, acc_sh.at[dstv.at[j + 1]], add=True)

  plsc.subcore_barrier()
  pltpu.sync_copy(acc_sh.at[pl.ds(s * stripe, stripe)],
                  out.at[pl.ds(c * 3 * NI + s * stripe, stripe)])


_inter_kernel = functools.partial(
    pl.kernel,
    out_type=jax.ShapeDtypeStruct((2 * 3 * NI,), jnp.float32),
    scratch_types=[
        pltpu.VMEM((CI, 128), jnp.int32),
        pltpu.VMEM((CI, 128), jnp.int32),
        pltpu.VMEM((128,), jnp.float32),
        pltpu.VMEM((128,), jnp.float32),
        pltpu.SemaphoreType.DMA,
        pltpu.SemaphoreType.DMA,
        pltpu.VMEM_SHARED((3 * NI,), jnp.float32),
    ],
)


_BLK = 1280


def _tc_pre_body(x_ref, pdeg_ref, w_ref, o_ref):
  deg = pdeg_ref[0, :] + pdeg_ref[1, :] + 1.0
  dis = lax.rsqrt(deg)
  m = jnp.dot(x_ref[...], w_ref[...], preferred_element_type=jnp.float32)
  o_ref[...] = dis[:, None] * m


def _tc_mid_body(acc_ref, pdeg_ref, b_ref, w_ref, o_ref):
  deg = pdeg_ref[0, :] + pdeg_ref[1, :] + 1.0
  dis = lax.rsqrt(deg)
  h = jnp.maximum(dis[:, None] * acc_ref[...] + b_ref[...], 0.0)
  m = jnp.dot(h, w_ref[...], preferred_element_type=jnp.float32)
  o_ref[...] = dis[:, None] * m


def _tc_pool_body(acc_ref, pdeg_ref, pdegi_ref, b_ref, o_ref, sums_ref):
  i = pl.program_id(0)
  deg = pdeg_ref[0, :] + pdeg_ref[1, :] + 1.0
  dis = lax.rsqrt(deg)
  h = jnp.maximum(dis[:, None] * acc_ref[...] + b_ref[...], 0.0)
  lrow = (i % 8) * _BLK + lax.broadcasted_iota(jnp.int32, (_BLK, 1), 0)
  h = jnp.where(lrow < N, h, 0.0)
  sums_ref[pl.ds(i, 1), :] = jnp.sum(h, axis=0, keepdims=True)

  @pl.when(i == 15)
  def _():
    allsums = sums_ref[...]
    h1m = jnp.sum(allsums[0:8], axis=0) * (1.0 / N)
    h2m = jnp.sum(allsums[8:16], axis=0) * (1.0 / N)
    degi = pdegi_ref[0] + pdegi_ref[1] + 1.0
    disi = lax.rsqrt(degi)
    outer = h1m[:, None] * h2m[None, :]
    o_ref[...] = disi * outer[None, :, :]


def _tc_head_body(ps_ref, g_ref, pdegi_ref, wi_ref, bi_ref, fcw_ref,
                  fcb_ref, o_ref, acc_ref):
  i = pl.program_id(0)

  @pl.when(i == 0)
  def _():
    acc_ref[...] = jnp.zeros((8, HID), jnp.float32)

  degi = pdegi_ref[0] + pdegi_ref[1] + 1.0
  disi = lax.rsqrt(degi)
  s = disi * (ps_ref[0] + ps_ref[1] + g_ref[...])
  for k in range(3):
    sk = s[k, :]
    contrib = jnp.maximum(sk[:, None] * wi_ref[k, :][None, :]
                          + bi_ref[k, :][None, :], 0.0)
    acc_ref[k, :] = acc_ref[k, :] + jnp.sum(contrib, axis=0)

  @pl.when(i == 15)
  def _():
    v = (acc_ref[0, :] + acc_ref[1, :] + acc_ref[2, :]) * (1.0 / (3.0 * NI))
    o_ref[...] = jnp.dot(v[None, :], fcw_ref[...],
                         preferred_element_type=jnp.float32) + fcb_ref[...]


def _pad_edges(idx, pad_val, total):
  pad = jnp.full((total - idx.shape[0],), pad_val, jnp.int32)
  return jnp.concatenate([idx.astype(jnp.int32), pad])


def kernel(x_lig, x_tar, W1, b1, W2, b2, Wi1, bi1, Wi2, bi2, Wi3, bi3,
           fcW, fcb, lig_e_idx, tar_e_idx, inter_idx1, inter_idx2,
           inter_idx3):
  f32 = jnp.float32
  mesh = _sc_mesh()

  x = jnp.concatenate([
      jnp.pad(x_lig.astype(f32), ((0, NP - N), (0, 0))),
      jnp.pad(x_tar.astype(f32), ((0, NP - N), (0, 0))),
  ])

  lig_src = lig_e_idx[0].astype(jnp.int32)
  lig_dst = lig_e_idx[1].astype(jnp.int32)
  tar_src = tar_e_idx[0].astype(jnp.int32)
  tar_dst = tar_e_idx[1].astype(jnp.int32)
  i_src = [e[0].astype(jnp.int32) for e in (inter_idx1, inter_idx2,
                                            inter_idx3)]
  i_dst = [e[1].astype(jnp.int32) for e in (inter_idx1, inter_idx2,
                                            inter_idx3)]

  conv_src = jnp.pad(jnp.concatenate([
      _pad_edges(lig_src, N, EPB),
      _pad_edges(tar_src + NP, NP + N, EPB),
  ]).reshape(2 * NS * CB, 128), ((0, 2 * IG), (0, 0)))
  conv_dst = jnp.pad(jnp.concatenate([
      _pad_edges(lig_dst, N, EPB),
      _pad_edges(tar_dst, N, EPB),
  ]).reshape(2 * NS * CB, 128), ((0, 2 * IG), (0, 0)))

  deg_bins = jnp.concatenate([
      lig_dst, tar_dst + NP,
      i_dst[0] + 2 * NP, i_dst[1] + 2 * NP + NI, i_dst[2] + 2 * NP + 2 * NI,
      jnp.full((DEG_EDGES - 2 * E - 3 * EI,), TRASH, jnp.int32),
  ]).reshape(32 * DEGC, 128)

  int_src = jnp.concatenate(
      [i_src[k] + k * NI for k in range(3)]).reshape(32 * CI, 128)
  int_dst = jnp.concatenate(
      [i_dst[k] + k * NI for k in range(3)]).reshape(32 * CI, 128)

  zeros_hbm = jnp.zeros((SD,), f32)

  pdeg_flat = _deg_kernel(_deg_body, mesh=mesh)(deg_bins, zeros_hbm)
  pdeg = pdeg_flat.reshape(2, BINS)
  pdeg_nodes = pdeg[:, :2 * NP]
  pdeg_inter = pdeg[:, 2 * NP:].reshape(2, 3, NI)
  pdeg_inter4 = pdeg_inter.reshape(2, 3, 128, 128)

  grid16 = 16
  spec_x = pl.BlockSpec((_BLK, HID), lambda i: (i, 0))
  spec_pdeg = pl.BlockSpec((2, _BLK), lambda i: (0, i))
  spec_w = pl.BlockSpec((HID, HID), lambda i: (0, 0))
  spec_b = pl.BlockSpec((1, HID), lambda i: (0, 0))

  g1 = pl.pallas_call(
      _tc_pre_body,
      grid=(grid16,),
      in_specs=[spec_x, spec_pdeg, spec_w],
      out_specs=spec_x,
      out_shape=jax.ShapeDtypeStruct((2 * NP, HID), f32),
  )(x, pdeg_nodes, W1.astype(f32))

  acc1 = _conv_kernel(_conv_body, mesh=mesh)(g1, conv_src, conv_dst)

  g2 = pl.pallas_call(
      _tc_mid_body,
      grid=(grid16,),
      in_specs=[spec_x, spec_pdeg, spec_b, spec_w],
      out_specs=spec_x,
      out_shape=jax.ShapeDtypeStruct((2 * NP, HID), f32),
  )(acc1, pdeg_nodes, b1.astype(f32).reshape(1, HID), W2.astype(f32))

  acc2 = _conv_kernel(_conv_body, mesh=mesh)(g2, conv_src, conv_dst)

  g_all = pl.pallas_call(
      _tc_pool_body,
      grid=(grid16,),
      in_specs=[
          spec_x, spec_pdeg,
          pl.BlockSpec((2, 3, 128, 128), lambda i: (0, 0, 0, 0)),
          spec_b,
      ],
      out_specs=pl.BlockSpec((3, 128, 128), lambda i: (0, 0, 0)),
      out_shape=jax.ShapeDtypeStruct((3, 128, 128), f32),
      scratch_shapes=[pltpu.VMEM((16, HID), f32)],
  )(acc2, pdeg_nodes, pdeg_inter4, b2.astype(f32).reshape(1, HID))
  g_flat = g_all.reshape(3 * NI)

  ps_flat = _inter_kernel(_inter_body, mesh=mesh)(
      g_flat, int_src, int_dst, zeros_hbm)
  ps = ps_flat.reshape(2, 3, NI)

  wi = jnp.concatenate([Wi1, Wi2, Wi3]).astype(f32)
  bi = jnp.stack([bi1, bi2, bi3]).astype(f32)
  ib = 1024
  y = pl.pallas_call(
      _tc_head_body,
      grid=(grid16,),
      in_specs=[
          pl.BlockSpec((2, 3, ib), lambda i: (0, 0, i)),
          pl.BlockSpec((3, ib), lambda i: (0, i)),
          pl.BlockSpec((2, 3, ib), lambda i: (0, 0, i)),
          pl.BlockSpec((3, HID), lambda i: (0, 0)),
          pl.BlockSpec((3, HID), lambda i: (0, 0)),
          pl.BlockSpec((HID, HID), lambda i: (0, 0)),
          pl.BlockSpec((1, HID), lambda i: (0, 0)),
      ],
      out_specs=pl.BlockSpec((1, HID), lambda i: (0, 0)),
      out_shape=jax.ShapeDtypeStruct((1, HID), f32),
      scratch_shapes=[pltpu.VMEM((8, HID), f32)],
  )(ps, g_flat.reshape(3, NI), pdeg_inter, wi, bi, fcW.astype(f32),
    fcb.astype(f32).reshape(1, HID))
  return y
